# baseline (speedup 1.0000x reference)
import functools

import jax
import jax.numpy as jnp
from jax import lax
from jax.experimental import pallas as pl
from jax.experimental.pallas import tpu as pltpu
from jax.experimental.pallas import tpu_sc as plsc

_B = 16384
_D0, _D1, _D2 = 32, 64, 128
_AGG = 128
_NC, _NS, _L = 2, 16, 16
_NW = _NC * _NS
_BPW = _B // _NW
_ICH = 128
_NCH = _BPW // _ICH


def _sc_mean_reduction(indexes2d, table0, table1, table2):
    mesh = plsc.VectorSubcoreMesh(core_axis_name="c", subcore_axis_name="s")

    @functools.partial(
        pl.kernel,
        mesh=mesh,
        out_type=jax.ShapeDtypeStruct((_B, _AGG), jnp.float32),
        compiler_params=pltpu.CompilerParams(use_tc_tiling_on_sc=False),
        scratch_types=[
            pltpu.VMEM((_NCH, _ICH), jnp.int32),
            pltpu.VMEM((_BPW, _D0), jnp.float32),
            pltpu.VMEM((_BPW, _D1), jnp.float32),
            pltpu.VMEM((_BPW, _D2), jnp.float32),
            pltpu.SemaphoreType.DMA,
        ],
    )
    def run(idx_hbm, t0_hbm, t1_hbm, t2_hbm, out_hbm, idx_v, b0, b1, acc, sem):
        wid = lax.axis_index("s") * _NC + lax.axis_index("c")
        base = wid * _BPW

        pltpu.sync_copy(idx_hbm.at[pl.ds(wid * _NCH, _NCH)], idx_v)

        copies = []
        for j in range(_NCH):
            ij = idx_v.at[j]
            rows = pl.ds(j * _ICH, _ICH)
            copies.append(pltpu.async_copy(t2_hbm.at[ij], acc.at[rows], sem))
            copies.append(pltpu.async_copy(t1_hbm.at[ij], b1.at[rows], sem))
            copies.append(pltpu.async_copy(t0_hbm.at[ij], b0.at[rows], sem))
        for c in copies:
            c.wait()

        third = jnp.float32(1.0 / 3.0)

        def body(r, carry):
            for j in range(_AGG // _L):
                cols = pl.ds(j * _L, _L)
                v = acc[r, cols]
                if j * _L < _D0:
                    v = v + b0[r, cols]
                if j * _L < _D1:
                    v = v + b1[r, cols]
                acc[r, cols] = v * third
            return carry

        lax.fori_loop(0, _BPW, body, 0)

        pltpu.sync_copy(acc, out_hbm.at[pl.ds(base, _BPW)])

    return run(indexes2d, table0, table1, table2)


def kernel(indexes, table0, table1, table2):
    idx2d = indexes.reshape(_NW * _NCH, _ICH)
    return _sc_mean_reduction(idx2d, table0, table1, table2)


# speedup vs baseline: 1.0827x; 1.0827x over previous
import functools

import jax
import jax.numpy as jnp
from jax import lax
from jax.experimental import pallas as pl
from jax.experimental.pallas import tpu as pltpu
from jax.experimental.pallas import tpu_sc as plsc

_B = 16384
_AGG = 128
_NC, _NS, _L = 2, 16, 16
_NW = _NC * _NS
_BPW = _B // _NW
_CH = 128
_NCH = _BPW // _CH
_NSET = 2


def _sc_mean_reduction(indexes2d, p0, p1, p2):
    mesh = plsc.VectorSubcoreMesh(core_axis_name="c", subcore_axis_name="s")

    bufs = []
    for _ in range(_NSET):
        bufs.extend([
            pltpu.VMEM((_CH, _AGG), jnp.float32),
            pltpu.VMEM((_CH, _AGG), jnp.float32),
            pltpu.VMEM((_CH, _AGG), jnp.float32),
        ])

    @functools.partial(
        pl.kernel,
        mesh=mesh,
        out_type=jax.ShapeDtypeStruct((_B, _AGG), jnp.float32),
        compiler_params=pltpu.CompilerParams(use_tc_tiling_on_sc=False),
        scratch_types=[pltpu.VMEM((_NCH, _CH), jnp.int32)]
        + bufs
        + [pltpu.SemaphoreType.DMA] * _NSET
        + [pltpu.SemaphoreType.DMA],
    )
    def run(idx_hbm, t0_hbm, t1_hbm, t2_hbm, out_hbm, idx_v, *scratch):
        gbufs = [scratch[s * 3:s * 3 + 3] for s in range(_NSET)]
        sems_in = scratch[_NSET * 3:_NSET * 3 + _NSET]
        sem_out = scratch[_NSET * 3 + _NSET]

        wid = lax.axis_index("s") * _NC + lax.axis_index("c")
        base = wid * _BPW

        pltpu.sync_copy(idx_hbm.at[pl.ds(wid * _NCH, _NCH)], idx_v)

        tables = (t0_hbm, t1_hbm, t2_hbm)
        in_handles = [None] * _NCH
        out_handles = [None] * _NCH

        def fire_in(c):
            s = c % _NSET
            ij = idx_v.at[c]
            in_handles[c] = [
                pltpu.async_copy(tables[t].at[ij], gbufs[s][t], sems_in[s])
                for t in range(3)
            ]

        third = jnp.float32(1.0 / 3.0)

        def combine(c):
            s = c % _NSET
            g0, g1, g2 = gbufs[s]

            def body(r, carry):
                for j in range(_AGG // _L):
                    cols = pl.ds(j * _L, _L)
                    g2[r, cols] = (g0[r, cols] + g1[r, cols] + g2[r, cols]) * third
                return carry

            lax.fori_loop(0, _CH, body, 0)

        fire_in(0)
        for c in range(_NCH):
            for h in in_handles[c]:
                h.wait()
            if c >= 1:
                out_handles[c - 1].wait()
            if c + 1 < _NCH:
                fire_in(c + 1)
            combine(c)
            out_handles[c] = pltpu.async_copy(
                gbufs[c % _NSET][2],
                out_hbm.at[pl.ds(base + c * _CH, _CH)],
                sem_out,
            )
        out_handles[_NCH - 1].wait()

    return run(indexes2d, p0, p1, p2)


def kernel(indexes, table0, table1, table2):
    idx2d = indexes.reshape(_NW * _NCH, _CH)
    p0 = jnp.pad(table0, ((0, 0), (0, _AGG - table0.shape[1])))
    p1 = jnp.pad(table1, ((0, 0), (0, _AGG - table1.shape[1])))
    return _sc_mean_reduction(idx2d, p0, p1, table2)
